# baseline (speedup 1.0000x reference)
import functools

import jax
import jax.numpy as jnp
from jax import lax
from jax.experimental import pallas as pl
from jax.experimental.pallas import tpu as pltpu
from jax.experimental.pallas import tpu_sc as plsc

B = 16384
D = 128
L = 16
NV = D // L
NC = 2
NS = 16
NW = NC * NS
BPW = B // NW
C = 128
NCHUNK = BPW // C
NG = C // L


def _rsqrt(s):
    i = plsc.bitcast(s, jnp.int32)
    i = jnp.int32(0x5F3759DF) - (i >> 1)
    y = plsc.bitcast(i, jnp.float32)
    for _ in range(3):
        y = y * (1.5 - 0.5 * s * y * y)
    return y


def _perm(x, idx):
    return x.at[idx].get(mode="promise_in_bounds")


def _merge(a, b, m, pk):
    return jnp.where(m, _perm(b, pk), a) + jnp.where(m, b, _perm(a, pk))


def _reduce16(vecs, lane):
    for k in (1, 2, 4, 8):
        m = (lane & k) != 0
        pk = lane ^ k
        vecs = [_merge(vecs[2 * i], vecs[2 * i + 1], m, pk)
                for i in range(len(vecs) // 2)]
    return vecs[0]


def _sc_kernel(h_hbm, t_hbm, r_hbm, tail_hbm, rel_hbm, out_hbm,
               hi_v, ti_v, ri_v, rows, pb, ib, outc, sems):
    wid = lax.axis_index("s") * NC + lax.axis_index("c")
    base = wid * BPW
    lane = lax.iota(jnp.int32, L)

    pltpu.sync_copy(h_hbm.at[pl.ds(base, BPW)], hi_v)
    pltpu.sync_copy(t_hbm.at[pl.ds(base, BPW)], ti_v)
    pltpu.sync_copy(r_hbm.at[pl.ds(base, BPW)], ri_v)

    def make_gathers(k, slot):
        return [
            pltpu.make_async_copy(tail_hbm.at[hi_v.at[pl.ds(k * C, C)]],
                                  rows.at[slot, 0], sems.at[slot]),
            pltpu.make_async_copy(tail_hbm.at[ti_v.at[pl.ds(k * C, C)]],
                                  rows.at[slot, 1], sems.at[slot]),
            pltpu.make_async_copy(rel_hbm.at[ri_v.at[pl.ds(k * C, C)]],
                                  rows.at[slot, 2], sems.at[slot]),
        ]

    for cp in make_gathers(0, 0):
        cp.start()

    def chunk_pair(kk, carry):
        for b in range(2):
            k = kk * 2 + b
            slot = b
            for cp in make_gathers(k, slot):
                cp.wait()
            if b == 0:
                for cp in make_gathers(k + 1, 1 - slot):
                    cp.start()
            else:
                @pl.when(kk != NCHUNK // 2 - 1)
                def _():
                    for cp in make_gathers(k + 1, 1 - slot):
                        cp.start()
            _compute_chunk(k, slot)
        return carry

    def _compute_chunk(k, slot):
        hrows = rows.at[slot, 0]
        trows = rows.at[slot, 1]
        rrows = rows.at[slot, 2]

        def z_body(g, carry):
            zero = jnp.zeros((L,), jnp.float32)
            for rbuf in range(3):
                pb[rbuf, pl.ds(g * L, L)] = zero
            outc[pl.ds(g * L, L)] = zero
            return carry

        lax.fori_loop(0, NG, z_body, 0, unroll=False)

        def p1_body(i, carry):
            idx_i = jnp.full((L,), i, jnp.int32)
            acc = [[None, None], [None, None], [None, None]]
            for j in range(NV):
                vs = [hrows[i, pl.ds(j * L, L)],
                      trows[i, pl.ds(j * L, L)],
                      rrows[i, pl.ds(j * L, L)]]
                for x in range(3):
                    q = vs[x] * vs[x]
                    a = acc[x][j & 1]
                    acc[x][j & 1] = q if a is None else a + q
            plsc.addupdate_scatter(pb.at[0], [idx_i], acc[0][0] + acc[0][1])
            plsc.addupdate_scatter(pb.at[1], [idx_i], acc[1][0] + acc[1][1])
            plsc.addupdate_scatter(pb.at[2], [idx_i], acc[2][0] + acc[2][1])
            return carry

        lax.fori_loop(0, C, p1_body, 0, unroll=4)

        def p2_body(g, carry):
            for rbuf in range(3):
                ssq = pb[rbuf, pl.ds(g * L, L)]
                ib[rbuf, g] = _rsqrt(jnp.maximum(ssq, 1e-24))
            return carry

        lax.fori_loop(0, NG, p2_body, 0, unroll=False)

        def p3_body(g, carry):
            ihv = ib[0, g]
            itv = ib[1, g]
            irv = ib[2, g]
            for m in range(L):
                i = g * L + m
                splat_m = jnp.full((L,), m, jnp.int32)
                ih = _perm(ihv, splat_m)
                it = _perm(itv, splat_m)
                ir = _perm(irv, splat_m)
                terms = [jnp.abs(hrows[i, pl.ds(j * L, L)] * ih
                                 + rrows[i, pl.ds(j * L, L)] * ir
                                 - trows[i, pl.ds(j * L, L)] * it)
                         for j in range(NV)]
                while len(terms) > 1:
                    terms = [terms[2 * n] + terms[2 * n + 1]
                             for n in range(len(terms) // 2)]
                plsc.addupdate_scatter(outc, [jnp.full((L,), i, jnp.int32)],
                                       terms[0])
            return carry

        lax.fori_loop(0, NG, p3_body, 0, unroll=False)

        pltpu.sync_copy(outc, out_hbm.at[pl.ds(base + k * C, C)])

    lax.fori_loop(0, NCHUNK // 2, chunk_pair, 0, unroll=False)


def kernel(batch_h, batch_t, batch_r, tail_emb, rel_emb):
    mesh = plsc.VectorSubcoreMesh(core_axis_name="c", subcore_axis_name="s")
    f = functools.partial(
        pl.kernel,
        mesh=mesh,
        out_type=jax.ShapeDtypeStruct((B,), jnp.float32),
        compiler_params=pltpu.CompilerParams(needs_layout_passes=False,
                                             use_tc_tiling_on_sc=False),
        scratch_types=[
            pltpu.VMEM((BPW,), jnp.int32),
            pltpu.VMEM((BPW,), jnp.int32),
            pltpu.VMEM((BPW,), jnp.int32),
            pltpu.VMEM((2, 3, C, D), jnp.float32),
            pltpu.VMEM((3, C), jnp.float32),
            pltpu.VMEM((3, NG, L), jnp.float32),
            pltpu.VMEM((C,), jnp.float32),
            pltpu.SemaphoreType.DMA((2,)),
        ],
    )(_sc_kernel)
    return f(batch_h.astype(jnp.int32), batch_t.astype(jnp.int32),
             batch_r.astype(jnp.int32), tail_emb, rel_emb)


# speedup vs baseline: 1.5194x; 1.5194x over previous
import functools

import jax
import jax.numpy as jnp
from jax import lax
from jax.experimental import pallas as pl
from jax.experimental.pallas import tpu as pltpu
from jax.experimental.pallas import tpu_sc as plsc

B = 16384
D = 128
L = 16
NV = D // L
NC = 2
NS = 16
NW = NC * NS
BPW = B // NW
C = 128
NCHUNK = BPW // C
NG = C // L


def _rsqrt(s):
    i = plsc.bitcast(s, jnp.int32)
    i = jnp.int32(0x5F3759DF) - (i >> 1)
    y = plsc.bitcast(i, jnp.float32)
    for _ in range(3):
        y = y * (1.5 - 0.5 * s * y * y)
    return y


def _perm(x, idx):
    return x.at[idx].get(mode="promise_in_bounds")


def _merge(a, b, m, pk):
    return jnp.where(m, _perm(b, pk), a) + jnp.where(m, b, _perm(a, pk))


def _reduce16(vecs, lane):
    for k in (1, 2, 4, 8):
        m = (lane & k) != 0
        pk = lane ^ k
        vecs = [_merge(vecs[2 * i], vecs[2 * i + 1], m, pk)
                for i in range(len(vecs) // 2)]
    return vecs[0]


def _sc_kernel(h_hbm, t_hbm, r_hbm, tail_hbm, rel_hbm, out_hbm,
               hi_v, ti_v, ri_v, rows, pb, ib, sb, outc, sems):
    wid = lax.axis_index("s") * NC + lax.axis_index("c")
    base = wid * BPW
    lane = lax.iota(jnp.int32, L)

    pltpu.sync_copy(h_hbm.at[pl.ds(base, BPW)], hi_v)
    pltpu.sync_copy(t_hbm.at[pl.ds(base, BPW)], ti_v)
    pltpu.sync_copy(r_hbm.at[pl.ds(base, BPW)], ri_v)

    def make_gathers(k, slot):
        return [
            pltpu.make_async_copy(tail_hbm.at[hi_v.at[pl.ds(k * C, C)]],
                                  rows.at[slot, 0], sems.at[slot]),
            pltpu.make_async_copy(tail_hbm.at[ti_v.at[pl.ds(k * C, C)]],
                                  rows.at[slot, 1], sems.at[slot]),
            pltpu.make_async_copy(rel_hbm.at[ri_v.at[pl.ds(k * C, C)]],
                                  rows.at[slot, 2], sems.at[slot]),
        ]

    for cp in make_gathers(0, 0):
        cp.start()

    def chunk_pair(kk, carry):
        for b in range(2):
            k = kk * 2 + b
            slot = b
            for cp in make_gathers(k, slot):
                cp.wait()
            if b == 0:
                for cp in make_gathers(k + 1, 1 - slot):
                    cp.start()
            else:
                @pl.when(kk != NCHUNK // 2 - 1)
                def _():
                    for cp in make_gathers(k + 1, 1 - slot):
                        cp.start()
            _compute_chunk(k, slot)
        return carry

    def _compute_chunk(k, slot):
        hrows = rows.at[slot, 0]
        trows = rows.at[slot, 1]
        rrows = rows.at[slot, 2]

        def p1_body(i, carry):
            acc = [[None, None], [None, None], [None, None]]
            for j in range(NV):
                vs = [hrows[i, pl.ds(j * L, L)],
                      trows[i, pl.ds(j * L, L)],
                      rrows[i, pl.ds(j * L, L)]]
                for x in range(3):
                    q = vs[x] * vs[x]
                    a = acc[x][j & 1]
                    acc[x][j & 1] = q if a is None else a + q
            pb[0, i] = acc[0][0] + acc[0][1]
            pb[1, i] = acc[1][0] + acc[1][1]
            pb[2, i] = acc[2][0] + acc[2][1]
            return carry

        lax.fori_loop(0, C, p1_body, 0, unroll=4)

        def p2_body(g, carry):
            for rbuf in range(3):
                vecs = [pb[rbuf, g * L + m] for m in range(L)]
                ssq = _reduce16(vecs, lane)
                ib[rbuf, g] = _rsqrt(jnp.maximum(ssq, 1e-24))
            return carry

        lax.fori_loop(0, NG, p2_body, 0, unroll=False)

        def p3_body(g, carry):
            ihv = ib[0, g]
            itv = ib[1, g]
            irv = ib[2, g]
            for m in range(L):
                i = g * L + m
                splat_m = jnp.full((L,), m, jnp.int32)
                ih = _perm(ihv, splat_m)
                it = _perm(itv, splat_m)
                ir = _perm(irv, splat_m)
                terms = [jnp.abs(hrows[i, pl.ds(j * L, L)] * ih
                                 + rrows[i, pl.ds(j * L, L)] * ir
                                 - trows[i, pl.ds(j * L, L)] * it)
                         for j in range(NV)]
                while len(terms) > 1:
                    terms = [terms[2 * n] + terms[2 * n + 1]
                             for n in range(len(terms) // 2)]
                sb[i] = terms[0]
            return carry

        lax.fori_loop(0, NG, p3_body, 0, unroll=False)

        def p4_body(g, carry):
            vecs = [sb[g * L + m] for m in range(L)]
            outc[pl.ds(g * L, L)] = _reduce16(vecs, lane)
            return carry

        lax.fori_loop(0, NG, p4_body, 0, unroll=False)

        pltpu.sync_copy(outc, out_hbm.at[pl.ds(base + k * C, C)])

    lax.fori_loop(0, NCHUNK // 2, chunk_pair, 0, unroll=False)


def kernel(batch_h, batch_t, batch_r, tail_emb, rel_emb):
    mesh = plsc.VectorSubcoreMesh(core_axis_name="c", subcore_axis_name="s")
    f = functools.partial(
        pl.kernel,
        mesh=mesh,
        out_type=jax.ShapeDtypeStruct((B,), jnp.float32),
        compiler_params=pltpu.CompilerParams(needs_layout_passes=False,
                                             use_tc_tiling_on_sc=False),
        scratch_types=[
            pltpu.VMEM((BPW,), jnp.int32),
            pltpu.VMEM((BPW,), jnp.int32),
            pltpu.VMEM((BPW,), jnp.int32),
            pltpu.VMEM((2, 3, C, D), jnp.float32),
            pltpu.VMEM((3, C, L), jnp.float32),
            pltpu.VMEM((3, NG, L), jnp.float32),
            pltpu.VMEM((C, L), jnp.float32),
            pltpu.VMEM((C,), jnp.float32),
            pltpu.SemaphoreType.DMA((2,)),
        ],
    )(_sc_kernel)
    return f(batch_h.astype(jnp.int32), batch_t.astype(jnp.int32),
             batch_r.astype(jnp.int32), tail_emb, rel_emb)


# speedup vs baseline: 1.6029x; 1.0550x over previous
import functools

import jax
import jax.numpy as jnp
from jax import lax
from jax.experimental import pallas as pl
from jax.experimental.pallas import tpu as pltpu
from jax.experimental.pallas import tpu_sc as plsc

B = 16384
D = 128
L = 16
NV = D // L
NC = 2
NS = 16
NW = NC * NS
BPW = B // NW
C = 128
NCHUNK = BPW // C
NG = C // L
NREL = 1000
NRELP = 1024
RPT = 64


def _rsqrt(s):
    i = plsc.bitcast(s, jnp.int32)
    i = jnp.int32(0x5F3759DF) - (i >> 1)
    y = plsc.bitcast(i, jnp.float32)
    for _ in range(3):
        y = y * (1.5 - 0.5 * s * y * y)
    return y


def _perm(x, idx):
    return x.at[idx].get(mode="promise_in_bounds")


def _merge(a, b, m, pk):
    return jnp.where(m, _perm(b, pk), a) + jnp.where(m, b, _perm(a, pk))


def _reduce16(vecs, lane):
    for k in (1, 2, 4, 8):
        m = (lane & k) != 0
        pk = lane ^ k
        vecs = [_merge(vecs[2 * i], vecs[2 * i + 1], m, pk)
                for i in range(len(vecs) // 2)]
    return vecs[0]


def _sc_kernel(h_hbm, t_hbm, r_hbm, tail_hbm, rel_hbm, out_hbm,
               hi_v, ti_v, ri_v, rows, pb, ib, outc, rtmp, invtmp, invrel,
               shinv, sems):
    wid = lax.axis_index("s") * NC + lax.axis_index("c")
    sid = lax.axis_index("s")
    base = wid * BPW
    lane = lax.iota(jnp.int32, L)

    pltpu.sync_copy(h_hbm.at[pl.ds(base, BPW)], hi_v)
    pltpu.sync_copy(t_hbm.at[pl.ds(base, BPW)], ti_v)
    pltpu.sync_copy(r_hbm.at[pl.ds(base, BPW)], ri_v)

    def make_gathers(k, slot):
        return [
            pltpu.make_async_copy(tail_hbm.at[hi_v.at[pl.ds(k * C, C)]],
                                  rows.at[slot, 0], sems.at[slot]),
            pltpu.make_async_copy(tail_hbm.at[ti_v.at[pl.ds(k * C, C)]],
                                  rows.at[slot, 1], sems.at[slot]),
            pltpu.make_async_copy(rel_hbm.at[ri_v.at[pl.ds(k * C, C)]],
                                  rows.at[slot, 2], sems.at[slot]),
        ]

    for cp in make_gathers(0, 0):
        cp.start()

    rstart = jnp.minimum(sid * RPT, NREL - RPT)
    pltpu.sync_copy(rel_hbm.at[pl.ds(rstart, RPT)], rtmp)

    def rel_p1(i, carry):
        v = [rtmp[i, pl.ds(j * L, L)] for j in range(NV)]
        sq = [a * a for a in v]
        while len(sq) > 1:
            sq = [sq[2 * n] + sq[2 * n + 1] for n in range(len(sq) // 2)]
        pb[0, i] = sq[0]
        return carry

    lax.fori_loop(0, RPT, rel_p1, 0, unroll=2)

    def rel_p2(g, carry):
        vecs = [pb[0, g * L + m] for m in range(L)]
        ssq = _reduce16(vecs, lane)
        invtmp[pl.ds(g * L, L)] = _rsqrt(jnp.maximum(ssq, 1e-24))
        return carry

    lax.fori_loop(0, RPT // L, rel_p2, 0, unroll=False)
    pltpu.sync_copy(invtmp, shinv.at[pl.ds(rstart, RPT)])
    plsc.subcore_barrier()
    pltpu.sync_copy(shinv, invrel)

    def chunk_pair(kk, carry):
        for b in range(2):
            k = kk * 2 + b
            slot = b
            for cp in make_gathers(k, slot):
                cp.wait()
            if b == 0:
                for cp in make_gathers(k + 1, 1 - slot):
                    cp.start()
            else:
                @pl.when(kk != NCHUNK // 2 - 1)
                def _():
                    for cp in make_gathers(k + 1, 1 - slot):
                        cp.start()
            _compute_chunk(k, slot)
        return carry

    def _compute_chunk(k, slot):
        hrows = rows.at[slot, 0]
        trows = rows.at[slot, 1]
        rrows = rows.at[slot, 2]

        def p1_body(i, carry):
            acc = [[None, None], [None, None]]
            for j in range(NV):
                vs = [hrows[i, pl.ds(j * L, L)],
                      trows[i, pl.ds(j * L, L)]]
                for x in range(2):
                    q = vs[x] * vs[x]
                    a = acc[x][j & 1]
                    acc[x][j & 1] = q if a is None else a + q
            pb[0, i] = acc[0][0] + acc[0][1]
            pb[1, i] = acc[1][0] + acc[1][1]
            return carry

        lax.fori_loop(0, C, p1_body, 0, unroll=4)

        def p2_body(g, carry):
            for rbuf in range(2):
                vecs = [pb[rbuf, g * L + m] for m in range(L)]
                ssq = _reduce16(vecs, lane)
                ib[rbuf, g] = _rsqrt(jnp.maximum(ssq, 1e-24))
            return carry

        lax.fori_loop(0, NG, p2_body, 0, unroll=False)

        def p3_body(g, carry):
            ihv = ib[0, g]
            itv = ib[1, g]
            ridx = ri_v[pl.ds(k * C + g * L, L)]
            irv = plsc.load_gather(invrel, [ridx])
            for m in range(L):
                i = g * L + m
                splat_m = jnp.full((L,), m, jnp.int32)
                ih = _perm(ihv, splat_m)
                it = _perm(itv, splat_m)
                ir = _perm(irv, splat_m)
                terms = [jnp.abs(hrows[i, pl.ds(j * L, L)] * ih
                                 + rrows[i, pl.ds(j * L, L)] * ir
                                 - trows[i, pl.ds(j * L, L)] * it)
                         for j in range(NV)]
                while len(terms) > 1:
                    terms = [terms[2 * n] + terms[2 * n + 1]
                             for n in range(len(terms) // 2)]
                pb[0, i] = terms[0]
            return carry

        lax.fori_loop(0, NG, p3_body, 0, unroll=False)

        def p4_body(g, carry):
            vecs = [pb[0, g * L + m] for m in range(L)]
            outc[pl.ds(g * L, L)] = _reduce16(vecs, lane)
            return carry

        lax.fori_loop(0, NG, p4_body, 0, unroll=False)

        pltpu.sync_copy(outc, out_hbm.at[pl.ds(base + k * C, C)])

    lax.fori_loop(0, NCHUNK // 2, chunk_pair, 0, unroll=False)


def kernel(batch_h, batch_t, batch_r, tail_emb, rel_emb):
    mesh = plsc.VectorSubcoreMesh(core_axis_name="c", subcore_axis_name="s")
    f = functools.partial(
        pl.kernel,
        mesh=mesh,
        out_type=jax.ShapeDtypeStruct((B,), jnp.float32),
        compiler_params=pltpu.CompilerParams(needs_layout_passes=False,
                                             use_tc_tiling_on_sc=False),
        scratch_types=[
            pltpu.VMEM((BPW,), jnp.int32),
            pltpu.VMEM((BPW,), jnp.int32),
            pltpu.VMEM((BPW,), jnp.int32),
            pltpu.VMEM((2, 3, C, D), jnp.float32),
            pltpu.VMEM((2, C, L), jnp.float32),
            pltpu.VMEM((2, NG, L), jnp.float32),
            pltpu.VMEM((C,), jnp.float32),
            pltpu.VMEM((RPT, D), jnp.float32),
            pltpu.VMEM((RPT,), jnp.float32),
            pltpu.VMEM((NRELP,), jnp.float32),
            pltpu.VMEM_SHARED((NRELP,), jnp.float32),
            pltpu.SemaphoreType.DMA((2,)),
        ],
    )(_sc_kernel)
    return f(batch_h.astype(jnp.int32), batch_t.astype(jnp.int32),
             batch_r.astype(jnp.int32), tail_emb, rel_emb)


# speedup vs baseline: 1.6957x; 1.0579x over previous
import functools

import jax
import jax.numpy as jnp
from jax import lax
from jax.experimental import pallas as pl
from jax.experimental.pallas import tpu as pltpu
from jax.experimental.pallas import tpu_sc as plsc

B = 16384
D = 128
L = 16
NV = D // L
NC = 2
NS = 16
NW = NC * NS
BPW = B // NW
C = 128
NCHUNK = BPW // C
NG = C // L
NREL = 1000
NRELP = 1024
RPT = 64


def _rsqrt(s):
    i = plsc.bitcast(s, jnp.int32)
    i = jnp.int32(0x5F3759DF) - (i >> 1)
    y = plsc.bitcast(i, jnp.float32)
    for _ in range(3):
        y = y * (1.5 - 0.5 * s * y * y)
    return y


def _perm(x, idx):
    return x.at[idx].get(mode="promise_in_bounds")


def _merge(a, b, m, pk):
    return jnp.where(m, _perm(b, pk), a) + jnp.where(m, b, _perm(a, pk))


def _reduce16(vecs, lane):
    for k in (1, 2, 4, 8):
        m = (lane & k) != 0
        pk = lane ^ k
        vecs = [_merge(vecs[2 * i], vecs[2 * i + 1], m, pk)
                for i in range(len(vecs) // 2)]
    return vecs[0]


def _sc_kernel(h_hbm, t_hbm, r_hbm, tail_hbm, rel_hbm, out_hbm,
               hi_v, ti_v, ri_v, rows, pb, ib, outc, rtmp, invtmp, invrel,
               shinv, sems):
    wid = lax.axis_index("s") * NC + lax.axis_index("c")
    sid = lax.axis_index("s")
    base = wid * BPW
    lane = lax.iota(jnp.int32, L)

    pltpu.sync_copy(h_hbm.at[pl.ds(base, BPW)], hi_v)
    pltpu.sync_copy(t_hbm.at[pl.ds(base, BPW)], ti_v)
    pltpu.sync_copy(r_hbm.at[pl.ds(base, BPW)], ri_v)

    def make_gathers(k, slot):
        return [
            pltpu.make_async_copy(tail_hbm.at[hi_v.at[pl.ds(k * C, C)]],
                                  rows.at[slot, 0], sems.at[slot]),
            pltpu.make_async_copy(tail_hbm.at[ti_v.at[pl.ds(k * C, C)]],
                                  rows.at[slot, 1], sems.at[slot]),
            pltpu.make_async_copy(rel_hbm.at[ri_v.at[pl.ds(k * C, C)]],
                                  rows.at[slot, 2], sems.at[slot]),
        ]

    for cp in make_gathers(0, 0):
        cp.start()

    rstart = jnp.minimum(sid * RPT, NREL - RPT)
    pltpu.sync_copy(rel_hbm.at[pl.ds(rstart, RPT)], rtmp)

    def rel_p1(i, carry):
        v = [rtmp[i, pl.ds(j * L, L)] for j in range(NV)]
        sq = [a * a for a in v]
        while len(sq) > 1:
            sq = [sq[2 * n] + sq[2 * n + 1] for n in range(len(sq) // 2)]
        pb[0, i] = sq[0]
        return carry

    lax.fori_loop(0, RPT, rel_p1, 0, unroll=2)

    def rel_p2(g, carry):
        vecs = [pb[0, g * L + m] for m in range(L)]
        ssq = _reduce16(vecs, lane)
        invtmp[pl.ds(g * L, L)] = _rsqrt(jnp.maximum(ssq, 1e-24))
        return carry

    lax.fori_loop(0, RPT // L, rel_p2, 0, unroll=False)
    pltpu.sync_copy(invtmp, shinv.at[pl.ds(rstart, RPT)])
    plsc.subcore_barrier()
    pltpu.sync_copy(shinv, invrel)

    def chunk_pair(kk, carry):
        for b in range(2):
            k = kk * 2 + b
            slot = b
            for cp in make_gathers(k, slot):
                cp.wait()
            if b == 0:
                for cp in make_gathers(k + 1, 1 - slot):
                    cp.start()
            else:
                @pl.when(kk != NCHUNK // 2 - 1)
                def _():
                    for cp in make_gathers(k + 1, 1 - slot):
                        cp.start()
            _compute_chunk(k, slot)
        return carry

    def _compute_chunk(k, slot):
        hrows = rows.at[slot, 0]
        trows = rows.at[slot, 1]
        rrows = rows.at[slot, 2]

        def p1_body(i, carry):
            acc = [[None, None], [None, None]]
            for j in range(NV):
                vs = [hrows[i, pl.ds(j * L, L)],
                      trows[i, pl.ds(j * L, L)]]
                for x in range(2):
                    q = vs[x] * vs[x]
                    a = acc[x][j & 1]
                    acc[x][j & 1] = q if a is None else a + q
            pb[0, i] = acc[0][0] + acc[0][1]
            pb[1, i] = acc[1][0] + acc[1][1]
            return carry

        lax.fori_loop(0, C, p1_body, 0, unroll=4)

        def p2_body(g, carry):
            for rbuf in range(2):
                vecs = [pb[rbuf, g * L + m] for m in range(L)]
                ssq = _reduce16(vecs, lane)
                ib[rbuf, g] = _rsqrt(jnp.maximum(ssq, 1e-24))
            return carry

        lax.fori_loop(0, NG, p2_body, 0, unroll=False)

        def p3_body(g, carry):
            ihv = ib[0, g]
            itv = ib[1, g]
            ridx = ri_v[pl.ds(k * C + g * L, L)]
            irv = plsc.load_gather(invrel, [ridx])
            for m in range(0, L, 2):
                tt = [None, None]
                for p in range(2):
                    splat_m = jnp.full((L,), m + p, jnp.int32)
                    tt[p] = (g * L + m + p, _perm(ihv, splat_m),
                             _perm(itv, splat_m), _perm(irv, splat_m), [])
                for j in range(NV):
                    for p in range(2):
                        i, ih, it, ir, terms = tt[p]
                        terms.append(jnp.abs(hrows[i, pl.ds(j * L, L)] * ih
                                             + rrows[i, pl.ds(j * L, L)] * ir
                                             - trows[i, pl.ds(j * L, L)] * it))
                for p in range(2):
                    i, ih, it, ir, terms = tt[p]
                    while len(terms) > 1:
                        terms = [terms[2 * n] + terms[2 * n + 1]
                                 for n in range(len(terms) // 2)]
                    pb[0, i] = terms[0]
            return carry

        lax.fori_loop(0, NG, p3_body, 0, unroll=False)

        def p4_body(g, carry):
            vecs = [pb[0, g * L + m] for m in range(L)]
            outc[pl.ds(g * L, L)] = _reduce16(vecs, lane)
            return carry

        lax.fori_loop(0, NG, p4_body, 0, unroll=False)

        pltpu.sync_copy(outc, out_hbm.at[pl.ds(base + k * C, C)])

    lax.fori_loop(0, NCHUNK // 2, chunk_pair, 0, unroll=False)


def kernel(batch_h, batch_t, batch_r, tail_emb, rel_emb):
    mesh = plsc.VectorSubcoreMesh(core_axis_name="c", subcore_axis_name="s")
    f = functools.partial(
        pl.kernel,
        mesh=mesh,
        out_type=jax.ShapeDtypeStruct((B,), jnp.float32),
        compiler_params=pltpu.CompilerParams(needs_layout_passes=False,
                                             use_tc_tiling_on_sc=False),
        scratch_types=[
            pltpu.VMEM((BPW,), jnp.int32),
            pltpu.VMEM((BPW,), jnp.int32),
            pltpu.VMEM((BPW,), jnp.int32),
            pltpu.VMEM((2, 3, C, D), jnp.float32),
            pltpu.VMEM((2, C, L), jnp.float32),
            pltpu.VMEM((2, NG, L), jnp.float32),
            pltpu.VMEM((C,), jnp.float32),
            pltpu.VMEM((RPT, D), jnp.float32),
            pltpu.VMEM((RPT,), jnp.float32),
            pltpu.VMEM((NRELP,), jnp.float32),
            pltpu.VMEM_SHARED((NRELP,), jnp.float32),
            pltpu.SemaphoreType.DMA((2,)),
        ],
    )(_sc_kernel)
    return f(batch_h.astype(jnp.int32), batch_t.astype(jnp.int32),
             batch_r.astype(jnp.int32), tail_emb, rel_emb)


# speedup vs baseline: 1.7268x; 1.0183x over previous
import functools

import jax
import jax.numpy as jnp
from jax import lax
from jax.experimental import pallas as pl
from jax.experimental.pallas import tpu as pltpu
from jax.experimental.pallas import tpu_sc as plsc

B = 16384
D = 128
L = 16
NV = D // L
NC = 2
NS = 16
NW = NC * NS
BPW = B // NW
C = 128
NCHUNK = BPW // C
NG = C // L
NREL = 1000
NRELP = 1024
RPT = 64


def _rsqrt(s):
    i = plsc.bitcast(s, jnp.int32)
    i = jnp.int32(0x5F3759DF) - (i >> 1)
    y = plsc.bitcast(i, jnp.float32)
    for _ in range(3):
        y = y * (1.5 - 0.5 * s * y * y)
    return y


def _perm(x, idx):
    return x.at[idx].get(mode="promise_in_bounds")


def _merge(a, b, m, pk):
    return jnp.where(m, _perm(b, pk), a) + jnp.where(m, b, _perm(a, pk))


def _reduce16(vecs, lane):
    for k in (1, 2, 4, 8):
        m = (lane & k) != 0
        pk = lane ^ k
        vecs = [_merge(vecs[2 * i], vecs[2 * i + 1], m, pk)
                for i in range(len(vecs) // 2)]
    return vecs[0]


def _sc_kernel(h_hbm, t_hbm, r_hbm, tail_hbm, rel_hbm, out_hbm,
               hi_v, ti_v, ri_v, rows, pb, ib, outc, rtmp, invtmp, invrel,
               shinv, sems):
    wid = lax.axis_index("s") * NC + lax.axis_index("c")
    sid = lax.axis_index("s")
    base = wid * BPW
    lane = lax.iota(jnp.int32, L)

    pltpu.sync_copy(h_hbm.at[pl.ds(base, BPW)], hi_v)
    pltpu.sync_copy(t_hbm.at[pl.ds(base, BPW)], ti_v)
    pltpu.sync_copy(r_hbm.at[pl.ds(base, BPW)], ri_v)

    def make_gathers(k, slot):
        return [
            pltpu.make_async_copy(tail_hbm.at[hi_v.at[pl.ds(k * C, C)]],
                                  rows.at[slot, 0], sems.at[slot]),
            pltpu.make_async_copy(tail_hbm.at[ti_v.at[pl.ds(k * C, C)]],
                                  rows.at[slot, 1], sems.at[slot]),
            pltpu.make_async_copy(rel_hbm.at[ri_v.at[pl.ds(k * C, C)]],
                                  rows.at[slot, 2], sems.at[slot]),
        ]

    for cp in make_gathers(0, 0):
        cp.start()

    rstart = jnp.minimum(sid * RPT, NREL - RPT)
    pltpu.sync_copy(rel_hbm.at[pl.ds(rstart, RPT)], rtmp)

    def rel_p1(i, carry):
        v = [rtmp[i, pl.ds(j * L, L)] for j in range(NV)]
        sq = [a * a for a in v]
        while len(sq) > 1:
            sq = [sq[2 * n] + sq[2 * n + 1] for n in range(len(sq) // 2)]
        pb[0, i] = sq[0]
        return carry

    lax.fori_loop(0, RPT, rel_p1, 0, unroll=2)

    def rel_p2(g, carry):
        vecs = [pb[0, g * L + m] for m in range(L)]
        ssq = _reduce16(vecs, lane)
        invtmp[pl.ds(g * L, L)] = _rsqrt(jnp.maximum(ssq, 1e-24))
        return carry

    lax.fori_loop(0, RPT // L, rel_p2, 0, unroll=False)
    pltpu.sync_copy(invtmp, shinv.at[pl.ds(rstart, RPT)])
    plsc.subcore_barrier()
    pltpu.sync_copy(shinv, invrel)

    def chunk_pair(kk, carry):
        for b in range(2):
            k = kk * 2 + b
            slot = b
            for cp in make_gathers(k, slot):
                cp.wait()
            if b == 0:
                for cp in make_gathers(k + 1, 1 - slot):
                    cp.start()
            else:
                @pl.when(kk != NCHUNK // 2 - 1)
                def _():
                    for cp in make_gathers(k + 1, 1 - slot):
                        cp.start()
            _compute_chunk(k, slot)
        return carry

    def _compute_chunk(k, slot):
        hrows = rows.at[slot, 0]
        trows = rows.at[slot, 1]
        rrows = rows.at[slot, 2]

        def p1_body(i, carry):
            acc = [[None] * 4, [None] * 4]
            for j in range(NV):
                vs = [hrows[i, pl.ds(j * L, L)],
                      trows[i, pl.ds(j * L, L)]]
                for x in range(2):
                    q = vs[x] * vs[x]
                    a = acc[x][j & 3]
                    acc[x][j & 3] = q if a is None else a + q
            pb[0, i] = (acc[0][0] + acc[0][1]) + (acc[0][2] + acc[0][3])
            pb[1, i] = (acc[1][0] + acc[1][1]) + (acc[1][2] + acc[1][3])
            return carry

        lax.fori_loop(0, C, p1_body, 0, unroll=4)

        def p2_body(g, carry):
            for rbuf in range(2):
                vecs = [pb[rbuf, g * L + m] for m in range(L)]
                ssq = _reduce16(vecs, lane)
                ib[rbuf, g] = _rsqrt(jnp.maximum(ssq, 1e-24))
            return carry

        lax.fori_loop(0, NG, p2_body, 0, unroll=False)

        def p3_body(g, carry):
            ihv = ib[0, g]
            itv = ib[1, g]
            ridx = ri_v[pl.ds(k * C + g * L, L)]
            irv = plsc.load_gather(invrel, [ridx])
            NI = 4
            for m in range(0, L, NI):
                tt = [None] * NI
                for p in range(NI):
                    splat_m = jnp.full((L,), m + p, jnp.int32)
                    tt[p] = (g * L + m + p, _perm(ihv, splat_m),
                             _perm(itv, splat_m), _perm(irv, splat_m), [])
                for j in range(NV):
                    for p in range(NI):
                        i, ih, it, ir, terms = tt[p]
                        terms.append(jnp.abs(hrows[i, pl.ds(j * L, L)] * ih
                                             + rrows[i, pl.ds(j * L, L)] * ir
                                             - trows[i, pl.ds(j * L, L)] * it))
                for p in range(NI):
                    i, ih, it, ir, terms = tt[p]
                    while len(terms) > 1:
                        terms = [terms[2 * n] + terms[2 * n + 1]
                                 for n in range(len(terms) // 2)]
                    pb[0, i] = terms[0]
            return carry

        lax.fori_loop(0, NG, p3_body, 0, unroll=False)

        def p4_body(g, carry):
            vecs = [pb[0, g * L + m] for m in range(L)]
            outc[pl.ds(g * L, L)] = _reduce16(vecs, lane)
            return carry

        lax.fori_loop(0, NG, p4_body, 0, unroll=False)

        pltpu.sync_copy(outc, out_hbm.at[pl.ds(base + k * C, C)])

    lax.fori_loop(0, NCHUNK // 2, chunk_pair, 0, unroll=False)


def kernel(batch_h, batch_t, batch_r, tail_emb, rel_emb):
    mesh = plsc.VectorSubcoreMesh(core_axis_name="c", subcore_axis_name="s")
    f = functools.partial(
        pl.kernel,
        mesh=mesh,
        out_type=jax.ShapeDtypeStruct((B,), jnp.float32),
        compiler_params=pltpu.CompilerParams(needs_layout_passes=False,
                                             use_tc_tiling_on_sc=False),
        scratch_types=[
            pltpu.VMEM((BPW,), jnp.int32),
            pltpu.VMEM((BPW,), jnp.int32),
            pltpu.VMEM((BPW,), jnp.int32),
            pltpu.VMEM((2, 3, C, D), jnp.float32),
            pltpu.VMEM((2, C, L), jnp.float32),
            pltpu.VMEM((2, NG, L), jnp.float32),
            pltpu.VMEM((C,), jnp.float32),
            pltpu.VMEM((RPT, D), jnp.float32),
            pltpu.VMEM((RPT,), jnp.float32),
            pltpu.VMEM((NRELP,), jnp.float32),
            pltpu.VMEM_SHARED((NRELP,), jnp.float32),
            pltpu.SemaphoreType.DMA((2,)),
        ],
    )(_sc_kernel)
    return f(batch_h.astype(jnp.int32), batch_t.astype(jnp.int32),
             batch_r.astype(jnp.int32), tail_emb, rel_emb)
